# Initial kernel scaffold; baseline (speedup 1.0000x reference)
import jax
import jax.numpy as jnp
from jax.experimental import pallas as pl


def kernel(x_customer, x_product, x_store, Wc, bc, Wp, bp, Ws, bs, Wl, bl, Wr, ln_g, ln_b, edge_index_buys, edge_index_bought_by, edge_index_visits, edge_index_visited_by, edge_index_sold_at, edge_index_sells):
    raise NotImplementedError("write your pallas kernel here")


# speedup vs baseline: 1.4535x; 1.4535x over previous
import functools

import jax
import jax.numpy as jnp
from jax import lax
from jax.experimental import pallas as pl
from jax.experimental.pallas import tpu as pltpu
from jax.experimental.pallas import tpu_sc as plsc

_H = 128
_NCE, _NPE, _NSE = 100000, 50000, 1000
_NCP, _NPP, _NSP = 100096, 50176, 1024
_EUNIT = 32768
_BIG = 1 << 30
_K = 4


def _ceil_to(x, m):
    return -(-x // m) * m


@functools.cache
def _make_segsum(ns_pad, ds, f, ep, racc):
    cr = ep // 4096
    ng = cr // _K
    rps = racc // 16
    nz = rps // 16
    mesh = plsc.VectorSubcoreMesh(core_axis_name="c", subcore_axis_name="s")

    scratch = [
        pltpu.VMEM((_K, 128), jnp.int32),
        pltpu.VMEM((_K, 128), jnp.int32),
        pltpu.VMEM((_K, 128, ds), jnp.float32),
        pltpu.VMEM((16, ds), jnp.float32),
        pltpu.VMEM_SHARED((racc + 16, ds), jnp.float32),
        pltpu.SemaphoreType.DMA,
    ]
    if f > 1:
        scratch.append(pltpu.VMEM((_K, 128), jnp.int32))

    @functools.partial(
        pl.kernel, mesh=mesh,
        out_type=jax.ShapeDtypeStruct((2 * f * racc, ds), jnp.float32),
        scratch_types=scratch,
        compiler_params=pltpu.CompilerParams(use_tc_tiling_on_sc=False),
    )
    def seg(table, src2d, off2d, out, srcbuf, offbuf, rows, zrow, acc, sem,
            *maybe_srcf):
        cid = lax.axis_index("c")
        sid = lax.axis_index("s")
        base_row = cid * (ep // 256) + sid * cr

        for r in range(16):
            for j in range(ds // 16):
                zrow[r, pl.ds(j * 16, 16)] = jnp.zeros((16,), jnp.float32)

        def pass_body(p, _):
            def zbody(k, _):
                pltpu.sync_copy(zrow, acc.at[pl.ds(sid * rps + k * 16, 16)])
                return 0
            lax.fori_loop(0, nz, zbody, 0)
            plsc.subcore_barrier()
            foff = p * ns_pad

            def gbody(g, _):
                rr = base_row + g * _K
                pltpu.sync_copy(src2d.at[pl.ds(rr, _K)], srcbuf)
                pltpu.sync_copy(off2d.at[pl.ds(rr, _K)], offbuf)
                if f > 1:
                    srcf = maybe_srcf[0]
                    for j in range(_K):
                        for v in range(8):
                            sl = pl.ds(v * 16, 16)
                            srcf[j, sl] = srcbuf[j, sl] + foff
                    srcuse = srcf
                else:
                    srcuse = srcbuf
                handles = []
                for j in range(_K):
                    handles.append(pltpu.async_copy(
                        table.at[srcuse.at[j]], rows.at[j], sem))
                for h in handles:
                    h.wait()
                for j in range(_K):
                    pltpu.sync_copy(rows.at[j], acc.at[offbuf.at[j]],
                                    add=True)
                return 0
            lax.fori_loop(0, ng, gbody, 0)
            plsc.subcore_barrier()

            pltpu.sync_copy(
                acc.at[pl.ds(sid * rps, rps)],
                out.at[pl.ds((cid * f + p) * racc + sid * rps, rps)])
            plsc.subcore_barrier()
            return 0

        lax.fori_loop(0, f, pass_body, 0)

    return seg


_BLK = 256


def _full(shape):
    return pl.BlockSpec(shape, lambda i: (0, 0))


def _rows(w):
    return pl.BlockSpec((_BLK, w), lambda i: (i, 0))


def _layer_tail(h, g, b):
    mu = jnp.mean(h, axis=1, keepdims=True)
    var = jnp.mean((h - mu) ** 2, axis=1, keepdims=True)
    hn = (h - mu) * lax.rsqrt(var + 1e-5) * g + b
    return jnp.maximum(hn, 0.0)


@functools.cache
def _make_stage_a(npad, dc1, dc2):
    def body(x16, a1c0, a1c1, a2c0, a2c1, aw1, aw2, bwh, cv, cst, g, b, o):
        a1 = a1c0[...] + a1c1[...]
        a2 = a2c0[...] + a2c1[...]
        c1 = a1[:, dc1:dc1 + 1]
        c2 = a2[:, dc2:dc2 + 1]
        na1 = a1 / jnp.maximum(c1, 1.0)
        na2 = a2 / jnp.maximum(c2, 1.0)
        m1 = (c1 > 0).astype(jnp.float32)
        m2 = (c2 > 0).astype(jnp.float32)
        h = (jnp.dot(na1, aw1[...], preferred_element_type=jnp.float32)
             + jnp.dot(na2, aw2[...], preferred_element_type=jnp.float32)
             + jnp.dot(x16[...], bwh[...], preferred_element_type=jnp.float32)
             + m1 * cv[0:1, :] + m2 * cv[1:2, :] + cst[...])
        o[...] = _layer_tail(h, g[...], b[...])

    return pl.pallas_call(
        body,
        grid=(npad // _BLK,),
        in_specs=[_rows(16)] * 5 + [_full((16, _H))] * 3
        + [_full((2, _H)), _full((1, _H)), _full((1, _H)), _full((1, _H))],
        out_specs=_rows(_H),
        out_shape=jax.ShapeDtypeStruct((npad, _H), jnp.float32),
    )


@functools.cache
def _make_stage_b(npad, dc1, dc2):
    def body(x, a1c0, a1c1, a2c0, a2c1, k1c0, k1c1, k2c0, k2c1,
             wl1, wl2, wrh, cst, g, b, o):
        c1 = (k1c0[...] + k1c1[...])[:, dc1:dc1 + 1]
        c2 = (k2c0[...] + k2c1[...])[:, dc2:dc2 + 1]
        na1 = (a1c0[...] + a1c1[...]) / jnp.maximum(c1, 1.0)
        na2 = (a2c0[...] + a2c1[...]) / jnp.maximum(c2, 1.0)
        h = (jnp.dot(na1, wl1[...], preferred_element_type=jnp.float32)
             + jnp.dot(na2, wl2[...], preferred_element_type=jnp.float32)
             + jnp.dot(x[...], wrh[...], preferred_element_type=jnp.float32)
             + cst[...])
        o[...] = _layer_tail(h, g[...], b[...])

    return pl.pallas_call(
        body,
        grid=(npad // _BLK,),
        in_specs=[_rows(_H)] * 5 + [_rows(16)] * 4 + [_full((_H, _H))] * 3
        + [_full((1, _H)), _full((1, _H)), _full((1, _H))],
        out_specs=_rows(_H),
        out_shape=jax.ShapeDtypeStruct((npad, _H), jnp.float32),
    )


_SRC_T = (0, 1, 0, 2, 1, 2)
_DST_T = (1, 0, 2, 0, 2, 1)
_DCOL = (6, 4, 3)
_INC = ((1, 3), (0, 5), (2, 4))
_L2DS = ((16, 8), (32, 4), (128, 1))


def _feat_split(x, ds, f):
    if f == 1:
        return x
    n = x.shape[0]
    return x.reshape(n, f, ds).transpose(1, 0, 2).reshape(f * n, ds)


def _unsplit(out, racc, ds, f):
    o = out.reshape(2, f, racc, ds)
    if f == 1:
        return o[0, 0], o[1, 0]
    return (o[0].transpose(1, 0, 2).reshape(racc, f * ds),
            o[1].transpose(1, 0, 2).reshape(racc, f * ds))


def kernel(x_customer, x_product, x_store, Wc, bc, Wp, bp, Ws, bs, Wl, bl, Wr,
           ln_g, ln_b, edge_index_buys, edge_index_bought_by,
           edge_index_visits, edge_index_visited_by, edge_index_sold_at,
           edge_index_sells):
    f32 = jnp.float32
    npads = (_NCP, _NPP, _NSP)
    nreal = (_NCE, _NPE, _NSE)

    def pad16(x, d, npad):
        o = jnp.zeros((npad, 16), f32)
        o = o.at[:x.shape[0], :d].set(x.astype(f32))
        return o.at[:x.shape[0], d].set(1.0)

    x16 = (pad16(x_customer, 6, _NCP), pad16(x_product, 4, _NPP),
           pad16(x_store, 3, _NSP))

    edges = (edge_index_buys, edge_index_bought_by, edge_index_visits,
             edge_index_visited_by, edge_index_sold_at, edge_index_sells)
    esrc, eoff, epad = [], [], []
    for e in range(6):
        ei = edges[e]
        n = ei.shape[1]
        ep = _ceil_to(n, _EUNIT)
        racc = npads[_DST_T[e]]
        s = jnp.concatenate([ei[0].astype(jnp.int32),
                             jnp.zeros((ep - n,), jnp.int32)])
        d = jnp.concatenate([ei[1].astype(jnp.int32),
                             jnp.full((ep - n,), _BIG, jnp.int32)])
        esrc.append(s.reshape(-1, 128))
        eoff.append(jnp.minimum(d, racc).reshape(-1, 128))
        epad.append(ep)

    wpad, bvec = [], (bc, bp, bs)
    for w, d in ((Wc, 6), (Wp, 4), (Ws, 3)):
        wpad.append(jnp.zeros((16, _H), f32).at[:d].set(w.astype(f32)))

    l1p = []
    for e in range(6):
        racc = npads[_DST_T[e]]
        seg = _make_segsum(npads[_SRC_T[e]], 16, 1, epad[e], racc)
        out = seg(x16[_SRC_T[e]], esrc[e], eoff[e])
        l1p.append(_unsplit(out, racc, 16, 1))

    xcur = []
    for t in range(3):
        e1, e2 = _INC[t]
        aw, cv, bw, cstv = [], [], [], []
        for e in (e1, e2):
            aw.append(0.5 * (wpad[_SRC_T[e]] @ Wl[0, e]))
            cv.append(0.5 * (bvec[_SRC_T[e]].astype(f32) @ Wl[0, e]))
            bw.append(0.5 * (wpad[t] @ Wr[0, e]))
            cstv.append(0.5 * (bl[0, e] + bvec[t].astype(f32) @ Wr[0, e]))
        stage = _make_stage_a(npads[t], _DCOL[_SRC_T[e1]], _DCOL[_SRC_T[e2]])
        xcur.append(stage(
            x16[t], l1p[e1][0], l1p[e1][1], l1p[e2][0], l1p[e2][1],
            aw[0], aw[1], bw[0] + bw[1], jnp.stack(cv),
            (cstv[0] + cstv[1]).reshape(1, _H),
            ln_g[0, t].reshape(1, _H), ln_b[0, t].reshape(1, _H)))

    l2p = []
    for e in range(6):
        t_dst = _DST_T[e]
        ds, fnum = _L2DS[t_dst]
        racc = npads[t_dst]
        ns_pad = npads[_SRC_T[e]]
        table = _feat_split(xcur[_SRC_T[e]], ds, fnum)
        seg = _make_segsum(ns_pad, ds, fnum, epad[e], racc)
        out = seg(table, esrc[e], eoff[e])
        l2p.append(_unsplit(out, racc, ds, fnum))

    res = []
    for t in range(3):
        e1, e2 = _INC[t]
        wl1 = 0.5 * Wl[1, e1]
        wl2 = 0.5 * Wl[1, e2]
        wrh = 0.5 * (Wr[1, e1] + Wr[1, e2])
        cstv = 0.5 * (bl[1, e1] + bl[1, e2])
        stage = _make_stage_b(npads[t], _DCOL[_SRC_T[e1]], _DCOL[_SRC_T[e2]])
        h = stage(
            xcur[t], l2p[e1][0], l2p[e1][1], l2p[e2][0], l2p[e2][1],
            l1p[e1][0], l1p[e1][1], l1p[e2][0], l1p[e2][1],
            wl1, wl2, wrh, cstv.reshape(1, _H),
            ln_g[1, t].reshape(1, _H), ln_b[1, t].reshape(1, _H))
        res.append(h[:nreal[t]])

    return tuple(res)
